# baseline (speedup 1.0000x reference)
import functools

import jax
import jax.numpy as jnp
from jax import lax
from jax.experimental import pallas as pl
from jax.experimental.pallas import tpu as pltpu
from jax.experimental.pallas import tpu_sc as plsc

VOCAB = 1000000
EMBED = 64
BATCH = 4096
HIST = 200

NUM_CORES = 2
NUM_SUBCORES = 16
NUM_WORKERS = NUM_CORES * NUM_SUBCORES
ROWS_PER_WORKER = BATCH // NUM_WORKERS
CHUNK_ROWS = 4
CHUNK_IDS = CHUNK_ROWS * HIST
NUM_CHUNKS = ROWS_PER_WORKER // CHUNK_ROWS
GATHER_MAX = 128
LANES = 16
VPR = EMBED // LANES

_splits = []
_off = 0
while _off < CHUNK_IDS:
    _sz = min(GATHER_MAX, CHUNK_IDS - _off)
    _splits.append((_off, _sz))
    _off += _sz
GATHER_SPLITS = tuple(_splits)


def _pool_kernel(ids_hbm, table_hbm, out_hbm, idx_v, rows_v, out_s, sem):
    wid = lax.axis_index("s") * NUM_CORES + lax.axis_index("c")
    worker_base = wid * ROWS_PER_WORKER

    @pl.loop(0, NUM_CHUNKS)
    def _chunk(c):
        row0 = worker_base + c * CHUNK_ROWS
        pltpu.sync_copy(ids_hbm.at[pl.ds(row0 * HIST, CHUNK_IDS)], idx_v)
        copies = []
        for off, sz in GATHER_SPLITS:
            copies.append(
                pltpu.async_copy(
                    table_hbm.at[idx_v.at[pl.ds(off, sz)]],
                    rows_v.at[pl.ds(off, sz)],
                    sem,
                )
            )
        for cp in copies:
            cp.wait()
        for r in range(CHUNK_ROWS):
            base = r * HIST

            @pl.loop(0, HIST, init_carry=tuple(
                jnp.zeros((LANES,), jnp.float32) for _ in range(VPR)))
            def _acc(i, carry):
                return tuple(
                    carry[v] + rows_v[base + i, pl.ds(v * LANES, LANES)]
                    for v in range(VPR)
                )

            for v in range(VPR):
                out_s[r, pl.ds(v * LANES, LANES)] = _acc[v] * (1.0 / HIST)
        pltpu.sync_copy(out_s, out_hbm.at[pl.ds(row0, CHUNK_ROWS)])


@jax.jit
def _pooled_means(ids_flat, emb_table):
    mesh = plsc.VectorSubcoreMesh(core_axis_name="c", subcore_axis_name="s")
    return pl.kernel(
        _pool_kernel,
        out_type=jax.ShapeDtypeStruct((BATCH, EMBED), jnp.float32),
        mesh=mesh,
        compiler_params=pltpu.CompilerParams(use_tc_tiling_on_sc=False),
        scratch_types=[
            pltpu.VMEM((CHUNK_IDS,), jnp.int32),
            pltpu.VMEM((CHUNK_IDS, EMBED), jnp.float32),
            pltpu.VMEM((CHUNK_ROWS, EMBED), jnp.float32),
            pltpu.SemaphoreType.DMA,
        ],
    )(ids_flat, emb_table)


def _proj_kernel(x_ref, w_ref, b_ref, o_ref):
    o_ref[...] = (
        lax.dot_general(
            x_ref[...], w_ref[...],
            (((1,), (1,)), ((), ())),
            preferred_element_type=jnp.float32,
        )
        + b_ref[...]
    )


@jax.jit
def _project(pooled, W, b2d):
    return pl.pallas_call(
        _proj_kernel,
        out_shape=jax.ShapeDtypeStruct((BATCH, EMBED), jnp.float32),
    )(pooled, W, b2d)


def kernel(input_ids, emb_table, W, b):
    ids_flat = input_ids.reshape(-1).astype(jnp.int32)
    pooled = _pooled_means(ids_flat, emb_table)
    return _project(pooled, W, b.reshape(1, EMBED))
